# baseline (speedup 1.0000x reference)
import functools

import jax
import jax.numpy as jnp
from jax import lax
from jax.experimental import pallas as pl
from jax.experimental.pallas import tpu as pltpu
from jax.experimental.pallas import tpu_sc as plsc

NUM_CHUNKS = 4
TOP_K = 2
CHUNK_IN = 1024
CHUNK_OUT = 1024

ROUTE_TILE = 512
MAIN_TILE_N = 512
COMB_TILE_O = 1024


def _route_kernel(x_ref, sums_ref, acc_ref):
    step = pl.program_id(0)
    nsteps = pl.num_programs(0)

    @pl.when(step == 0)
    def _init():
        acc_ref[...] = jnp.zeros_like(acc_ref)

    xv = x_ref[...]
    a = jnp.abs(xv).reshape(ROUTE_TILE, NUM_CHUNKS, CHUNK_IN // 128, 128)
    acc_ref[...] += jnp.sum(a, axis=0)

    @pl.when(step == nsteps - 1)
    def _emit():
        for c in range(NUM_CHUNKS):
            s = jnp.sum(acc_ref[c])
            for j in range(16):
                sums_ref[c * 16 + j] = s


def _route(xc):
    n = xc.shape[0]
    n_tiles = n // ROUTE_TILE
    return pl.pallas_call(
        _route_kernel,
        grid=(n_tiles,),
        in_specs=[pl.BlockSpec((ROUTE_TILE, NUM_CHUNKS, CHUNK_IN),
                               lambda i: (i, 0, 0))],
        out_specs=pl.BlockSpec(memory_space=pltpu.SMEM),
        out_shape=jax.ShapeDtypeStruct((NUM_CHUNKS * 16,), jnp.float32),
        scratch_shapes=[
            pltpu.VMEM((NUM_CHUNKS, CHUNK_IN // 128, 128), jnp.float32)],
    )(xc)


def _sc_top2(sums):
    mesh = plsc.VectorSubcoreMesh(core_axis_name="c", subcore_axis_name="s")

    @functools.partial(
        pl.kernel, mesh=mesh,
        out_type=jax.ShapeDtypeStruct((16,), jnp.int32),
        scratch_types=[
            pltpu.VMEM((NUM_CHUNKS * 16,), jnp.float32),
            pltpu.VMEM((16,), jnp.int32),
        ],
    )
    def top2(sums_hbm, idx_hbm, sums_v, idx_v):
        first = jnp.logical_and(lax.axis_index("c") == 0,
                                lax.axis_index("s") == 0)

        @pl.when(first)
        def _():
            pltpu.sync_copy(sums_hbm, sums_v)
            k = [sums_v[pl.ds(c * 16, 16)] for c in range(NUM_CHUNKS)]
            io = lax.iota(jnp.int32, 16)
            m1 = jnp.maximum(jnp.maximum(k[0], k[1]),
                             jnp.maximum(k[2], k[3]))
            i1 = jnp.where(k[0] == m1, 0,
                           jnp.where(k[1] == m1, 1,
                                     jnp.where(k[2] == m1, 2, 3)))
            i1 = i1.astype(jnp.int32)
            k2 = [jnp.where(i1 == c, jnp.float32(-2.0), k[c])
                  for c in range(NUM_CHUNKS)]
            m2 = jnp.maximum(jnp.maximum(k2[0], k2[1]),
                             jnp.maximum(k2[2], k2[3]))
            i2 = jnp.where(k2[0] == m2, 0,
                           jnp.where(k2[1] == m2, 1,
                                     jnp.where(k2[2] == m2, 2, 3)))
            i2 = i2.astype(jnp.int32)
            idx_v[...] = jnp.where(io == 0, i1,
                                   jnp.where(io == 1, i2, 0))
            pltpu.sync_copy(idx_v, idx_hbm)

    return top2(sums)


def _combine_kernel(idx_ref, w0_ref, wf_ref, b0_ref, bf_ref, wc_ref, be_ref):
    k = pl.program_id(1)
    wc_ref[...] = jax.lax.dot_general(
        w0_ref[0], wf_ref[0], (((1,), (0,)), ((), ())),
        preferred_element_type=jnp.float32)
    part = jax.lax.dot_general(
        b0_ref[0], wf_ref[0], (((1,), (0,)), ((), ())),
        preferred_element_type=jnp.float32)

    @pl.when(k == 0)
    def _first():
        be_ref[...] = bf_ref[...] + part

    @pl.when(k != 0)
    def _rest():
        be_ref[...] += part


def _combine(idx, W0, Wfk, b03, bf2):
    o_tiles = Wfk.shape[2] // COMB_TILE_O
    grid_spec = pltpu.PrefetchScalarGridSpec(
        num_scalar_prefetch=1,
        grid=(o_tiles, TOP_K),
        in_specs=[
            pl.BlockSpec((1, CHUNK_IN, CHUNK_OUT),
                         lambda o, k, idx: (idx[k], 0, 0)),
            pl.BlockSpec((1, CHUNK_OUT, COMB_TILE_O),
                         lambda o, k, idx: (k, 0, o)),
            pl.BlockSpec((1, 1, CHUNK_OUT),
                         lambda o, k, idx: (idx[k], 0, 0)),
            pl.BlockSpec((1, COMB_TILE_O),
                         lambda o, k, idx: (0, o)),
        ],
        out_specs=[
            pl.BlockSpec((CHUNK_OUT, COMB_TILE_O),
                         lambda o, k, idx: (k, o)),
            pl.BlockSpec((1, COMB_TILE_O),
                         lambda o, k, idx: (0, o)),
        ],
    )
    return pl.pallas_call(
        _combine_kernel,
        grid_spec=grid_spec,
        out_shape=[
            jax.ShapeDtypeStruct((TOP_K * CHUNK_OUT, Wfk.shape[2]), jnp.float32),
            jax.ShapeDtypeStruct((1, Wfk.shape[2]), jnp.float32),
        ],
        compiler_params=pltpu.CompilerParams(
            dimension_semantics=("parallel", "arbitrary")),
    )(idx, W0, Wfk, b03, bf2)


def _main_kernel(idx_ref, x0_ref, x1_ref, wc_ref, be_ref, out_ref):
    dims = (((1,), (0,)), ((), ()))
    out_ref[...] = jax.lax.dot_general(
        x0_ref[...], wc_ref[:CHUNK_OUT, :], dims,
        preferred_element_type=jnp.float32)
    out_ref[...] += jax.lax.dot_general(
        x1_ref[...], wc_ref[CHUNK_OUT:, :], dims,
        preferred_element_type=jnp.float32)
    out_ref[...] += be_ref[...]


def _main(idx, xb, Wc, be):
    n = xb.shape[0]
    out_f = Wc.shape[1]
    xb2 = xb
    grid_spec = pltpu.PrefetchScalarGridSpec(
        num_scalar_prefetch=1,
        grid=(n // MAIN_TILE_N,),
        in_specs=[
            pl.BlockSpec((MAIN_TILE_N, CHUNK_IN),
                         lambda i, idx: (i, idx[0])),
            pl.BlockSpec((MAIN_TILE_N, CHUNK_IN),
                         lambda i, idx: (i, idx[1])),
            pl.BlockSpec((TOP_K * CHUNK_OUT, out_f), lambda i, idx: (0, 0)),
            pl.BlockSpec((1, out_f), lambda i, idx: (0, 0)),
        ],
        out_specs=pl.BlockSpec((MAIN_TILE_N, out_f), lambda i, idx: (i, 0)),
    )
    return pl.pallas_call(
        _main_kernel,
        grid_spec=grid_spec,
        out_shape=jax.ShapeDtypeStruct((n, out_f), jnp.float32),
        compiler_params=pltpu.CompilerParams(
            dimension_semantics=("arbitrary",)),
    )(idx, xb2, xb2, Wc, be)


def kernel(x, W0, b0, Wf, bf):
    n = x.shape[0]
    xc = x.reshape(n, NUM_CHUNKS, CHUNK_IN)
    sums = _route(xc)
    idx = _sc_top2(sums)
    Wfk = Wf.reshape(TOP_K, CHUNK_OUT, -1)
    b03 = b0.reshape(NUM_CHUNKS, 1, CHUNK_OUT)
    bf2 = bf.reshape(1, -1)
    Wc, be = _combine(idx, W0, Wfk, b03, bf2)
    return _main(idx, x, Wc, be)


# speedup vs baseline: 1.0059x; 1.0059x over previous
import functools

import jax
import jax.numpy as jnp
from jax import lax
from jax.experimental import pallas as pl
from jax.experimental.pallas import tpu as pltpu
from jax.experimental.pallas import tpu_sc as plsc

NUM_CHUNKS = 4
TOP_K = 2
CHUNK_IN = 1024
CHUNK_OUT = 1024

ROUTE_TILE = 512
MAIN_TILE_N = 512
COMB_TILE_O = 2048


def _route_kernel(x_ref, sums_ref, acc_ref):
    step = pl.program_id(0)
    nsteps = pl.num_programs(0)

    @pl.when(step == 0)
    def _init():
        acc_ref[...] = jnp.zeros_like(acc_ref)

    xv = x_ref[...]
    a = jnp.abs(xv).reshape(ROUTE_TILE, NUM_CHUNKS, CHUNK_IN // 128, 128)
    acc_ref[...] += jnp.sum(a, axis=0)

    @pl.when(step == nsteps - 1)
    def _emit():
        for c in range(NUM_CHUNKS):
            s = jnp.sum(acc_ref[c])
            for j in range(16):
                sums_ref[c * 16 + j] = s


def _route(xc):
    n = xc.shape[0]
    n_tiles = n // ROUTE_TILE
    return pl.pallas_call(
        _route_kernel,
        grid=(n_tiles,),
        in_specs=[pl.BlockSpec((ROUTE_TILE, NUM_CHUNKS, CHUNK_IN),
                               lambda i: (i, 0, 0))],
        out_specs=pl.BlockSpec(memory_space=pltpu.SMEM),
        out_shape=jax.ShapeDtypeStruct((NUM_CHUNKS * 16,), jnp.float32),
        scratch_shapes=[
            pltpu.VMEM((NUM_CHUNKS, CHUNK_IN // 128, 128), jnp.float32)],
    )(xc)


def _sc_top2(sums):
    mesh = plsc.VectorSubcoreMesh(core_axis_name="c", subcore_axis_name="s")

    @functools.partial(
        pl.kernel, mesh=mesh,
        out_type=jax.ShapeDtypeStruct((16,), jnp.int32),
        scratch_types=[
            pltpu.VMEM((NUM_CHUNKS * 16,), jnp.float32),
            pltpu.VMEM((16,), jnp.int32),
        ],
    )
    def top2(sums_hbm, idx_hbm, sums_v, idx_v):
        first = jnp.logical_and(lax.axis_index("c") == 0,
                                lax.axis_index("s") == 0)

        @pl.when(first)
        def _():
            pltpu.sync_copy(sums_hbm, sums_v)
            k = [sums_v[pl.ds(c * 16, 16)] for c in range(NUM_CHUNKS)]
            io = lax.iota(jnp.int32, 16)
            m1 = jnp.maximum(jnp.maximum(k[0], k[1]),
                             jnp.maximum(k[2], k[3]))
            i1 = jnp.where(k[0] == m1, 0,
                           jnp.where(k[1] == m1, 1,
                                     jnp.where(k[2] == m1, 2, 3)))
            i1 = i1.astype(jnp.int32)
            k2 = [jnp.where(i1 == c, jnp.float32(-2.0), k[c])
                  for c in range(NUM_CHUNKS)]
            m2 = jnp.maximum(jnp.maximum(k2[0], k2[1]),
                             jnp.maximum(k2[2], k2[3]))
            i2 = jnp.where(k2[0] == m2, 0,
                           jnp.where(k2[1] == m2, 1,
                                     jnp.where(k2[2] == m2, 2, 3)))
            i2 = i2.astype(jnp.int32)
            idx_v[...] = jnp.where(io == 0, i1,
                                   jnp.where(io == 1, i2, 0))
            pltpu.sync_copy(idx_v, idx_hbm)

    return top2(sums)


def _combine_kernel(idx_ref, w0_ref, wf_ref, b0_ref, bf_ref, wc_ref, be_ref):
    k = pl.program_id(1)
    wc_ref[...] = jax.lax.dot_general(
        w0_ref[0], wf_ref[0], (((1,), (0,)), ((), ())),
        preferred_element_type=jnp.float32)
    part = jax.lax.dot_general(
        b0_ref[0], wf_ref[0], (((1,), (0,)), ((), ())),
        preferred_element_type=jnp.float32)

    @pl.when(k == 0)
    def _first():
        be_ref[...] = bf_ref[...] + part

    @pl.when(k != 0)
    def _rest():
        be_ref[...] += part


def _combine(idx, W0, Wfk, b03, bf2):
    o_tiles = Wfk.shape[2] // COMB_TILE_O
    grid_spec = pltpu.PrefetchScalarGridSpec(
        num_scalar_prefetch=1,
        grid=(o_tiles, TOP_K),
        in_specs=[
            pl.BlockSpec((1, CHUNK_IN, CHUNK_OUT),
                         lambda o, k, idx: (idx[k], 0, 0)),
            pl.BlockSpec((1, CHUNK_OUT, COMB_TILE_O),
                         lambda o, k, idx: (k, 0, o)),
            pl.BlockSpec((1, 1, CHUNK_OUT),
                         lambda o, k, idx: (idx[k], 0, 0)),
            pl.BlockSpec((1, COMB_TILE_O),
                         lambda o, k, idx: (0, o)),
        ],
        out_specs=[
            pl.BlockSpec((CHUNK_OUT, COMB_TILE_O),
                         lambda o, k, idx: (k, o)),
            pl.BlockSpec((1, COMB_TILE_O),
                         lambda o, k, idx: (0, o)),
        ],
    )
    return pl.pallas_call(
        _combine_kernel,
        grid_spec=grid_spec,
        out_shape=[
            jax.ShapeDtypeStruct((TOP_K * CHUNK_OUT, Wfk.shape[2]), jnp.float32),
            jax.ShapeDtypeStruct((1, Wfk.shape[2]), jnp.float32),
        ],
        compiler_params=pltpu.CompilerParams(
            dimension_semantics=("parallel", "arbitrary")),
    )(idx, W0, Wfk, b03, bf2)


def _main_kernel(idx_ref, x0_ref, x1_ref, wc_ref, be_ref, out_ref):
    dims = (((1,), (0,)), ((), ()))
    out_ref[...] = jax.lax.dot_general(
        x0_ref[...], wc_ref[:CHUNK_OUT, :], dims,
        preferred_element_type=jnp.float32)
    out_ref[...] += jax.lax.dot_general(
        x1_ref[...], wc_ref[CHUNK_OUT:, :], dims,
        preferred_element_type=jnp.float32)
    out_ref[...] += be_ref[...]


def _main(idx, xb, Wc, be):
    n = xb.shape[0]
    out_f = Wc.shape[1]
    grid_spec = pltpu.PrefetchScalarGridSpec(
        num_scalar_prefetch=1,
        grid=(n // MAIN_TILE_N,),
        in_specs=[
            pl.BlockSpec((MAIN_TILE_N, CHUNK_IN),
                         lambda i, idx: (i, idx[0])),
            pl.BlockSpec((MAIN_TILE_N, CHUNK_IN),
                         lambda i, idx: (i, idx[1])),
            pl.BlockSpec((TOP_K * CHUNK_OUT, out_f), lambda i, idx: (0, 0)),
            pl.BlockSpec((1, out_f), lambda i, idx: (0, 0)),
        ],
        out_specs=pl.BlockSpec((MAIN_TILE_N, out_f), lambda i, idx: (i, 0)),
    )
    return pl.pallas_call(
        _main_kernel,
        grid_spec=grid_spec,
        out_shape=jax.ShapeDtypeStruct((n, out_f), jnp.float32),
        compiler_params=pltpu.CompilerParams(
            dimension_semantics=("arbitrary",)),
    )(idx, xb, xb, Wc, be)


def kernel(x, W0, b0, Wf, bf):
    n = x.shape[0]
    xc = x.reshape(n, NUM_CHUNKS, CHUNK_IN)
    sums = _route(xc)
    idx = _sc_top2(sums)
    Wfk = Wf.reshape(TOP_K, CHUNK_OUT, -1)
    b03 = b0.reshape(NUM_CHUNKS, 1, CHUNK_OUT)
    bf2 = bf.reshape(1, -1)
    Wc, be = _combine(idx, W0, Wfk, b03, bf2)
    return _main(idx, x, Wc, be)
